# baseline (speedup 1.0000x reference)
import functools

import jax
import jax.numpy as jnp
from jax import lax
from jax.experimental import pallas as pl
from jax.experimental.pallas import tpu as pltpu
from jax.experimental.pallas import tpu_sc as plsc

D = 2048
T = 2048
B = 2
N = 8192
EPS = 1e-8

NC = 2
NS = 16
NW = NC * NS
L = 16

GROUPS = N // L
GPW = GROUPS // NW


def _sc_body(zt_hbm, r_hbm, ii_hbm, jj_hbm, num_hbm, s_hbm,
             ii_v, jj_v, ridx_v, r_v, rows_i, rows_j, o_v, s_v, sem):
  wid = lax.axis_index("s") * NC + lax.axis_index("c")
  lanes = lax.iota(jnp.int32, L)

  def group_body(gl, carry0):
    g = wid * GPW + gl
    pbase = g * L
    pltpu.sync_copy(ii_hbm.at[pl.ds(pbase, L)], ii_v)
    pltpu.sync_copy(jj_hbm.at[pl.ds(pbase, L)], jj_v)
    pltpu.sync_copy(r_hbm.at[pl.ds(pbase, L)], r_v)
    d = jnp.exp(-r_v[...])

    for b in range(B):
      ridx_v[...] = ii_v[...] + b * D
      pltpu.async_copy(zt_hbm.at[ridx_v], rows_i, sem).wait()
      ridx_v[...] = jj_v[...] + b * D
      pltpu.async_copy(zt_hbm.at[ridx_v], rows_j, sem).wait()

      def t_body(tt, carry):
        w, acc, ssum, tvec = carry
        zi = plsc.load_gather(rows_i, [lanes, tvec])
        zj = plsc.load_gather(rows_j, [lanes, tvec])
        acc = acc + zi * zj * w
        ssum = ssum + w
        w = w * d
        tvec = tvec - 1
        return w, acc, ssum, tvec

      init = (jnp.ones((L,), jnp.float32),
              jnp.zeros((L,), jnp.float32),
              jnp.zeros((L,), jnp.float32),
              jnp.full((L,), T - 1, jnp.int32))
      res = lax.fori_loop(0, T, t_body, init)
      acc, ssum = res[1], res[2]

      o_v[...] = acc
      pltpu.sync_copy(o_v, num_hbm.at[b, pl.ds(pbase, L)])
      if b == 0:
        s_v[...] = ssum
        pltpu.sync_copy(s_v, s_hbm.at[pl.ds(pbase, L)])
    return carry0

  lax.fori_loop(0, GPW, group_body, None)


_sc_call = functools.partial(
    pl.kernel,
    mesh=plsc.VectorSubcoreMesh(core_axis_name="c", subcore_axis_name="s"),
    compiler_params=pltpu.CompilerParams(
        use_tc_tiling_on_sc=False, needs_layout_passes=False),
    out_type=[jax.ShapeDtypeStruct((B, N), jnp.float32),
              jax.ShapeDtypeStruct((N,), jnp.float32)],
    scratch_types=[
        pltpu.VMEM((L,), jnp.int32),
        pltpu.VMEM((L,), jnp.int32),
        pltpu.VMEM((L,), jnp.int32),
        pltpu.VMEM((L,), jnp.float32),
        pltpu.VMEM((L, T), jnp.float32),
        pltpu.VMEM((L, T), jnp.float32),
        pltpu.VMEM((L,), jnp.float32),
        pltpu.VMEM((L,), jnp.float32),
        pltpu.SemaphoreType.DMA,
    ],
)(_sc_body)


@jax.jit
def kernel(z_hist, decay_rates, idx_i, idx_j):
  zt = jnp.transpose(z_hist, (0, 2, 1)).reshape(B * D, T)
  r = jax.nn.softplus(decay_rates)
  num, s = _sc_call(zt, r, idx_i.astype(jnp.int32), idx_j.astype(jnp.int32))
  return num / jnp.sqrt(s + EPS)[None, :]
